# baseline (speedup 1.0000x reference)
import jax
import jax.numpy as jnp
from jax.experimental import pallas as pl
from jax.experimental.pallas import tpu as pltpu

N = 325
D = 64
DFF = 128
KSTEPS = 64
NB = 4


def _prep(cirmat_ref, bn_g_ref, bn_b_ref, pe_W_ref, pe_b_ref, W1_ref,
          b1_ref, S_ref, c1T_ref, posT_ref):
    A = cirmat_ref[...]
    rows = jax.lax.broadcasted_iota(jnp.int32, (N, N), 0)
    cols = jax.lax.broadcasted_iota(jnp.int32, (N, N), 1)
    eye = (rows == cols).astype(jnp.float32)

    deg = jnp.sum(A, axis=1)
    dinv = jnp.where(deg > 0, 1.0 / deg, 0.0)
    P = dinv[:, None] * A
    P2 = jnp.dot(P, P, preferred_element_type=jnp.float32)
    P3 = jnp.dot(P2, P, preferred_element_type=jnp.float32)
    P4 = jnp.dot(P2, P2, preferred_element_type=jnp.float32)
    PT = jnp.transpose(P)
    P2T = jnp.transpose(P2)
    P3T = jnp.transpose(P3)
    posT_ref[0:3, :] = jnp.stack([jnp.sum(P * eye, axis=0),
                                  jnp.sum(P2 * eye, axis=0),
                                  jnp.sum(P3 * eye, axis=0)])

    R = P4
    for j in range(1, 17):
        quad = jnp.stack([jnp.sum(eye * R, axis=0),
                          jnp.sum(PT * R, axis=0),
                          jnp.sum(P2T * R, axis=0),
                          jnp.sum(P3T * R, axis=0)])
        posT_ref[4 * j - 1:4 * j + 3, :] = quad
        if j < 16:
            R = jnp.dot(R, P4, preferred_element_type=jnp.float32)
    posT = posT_ref[0:KSTEPS, :]

    mean = jnp.mean(posT, axis=1, keepdims=True)
    var = jnp.mean((posT - mean) ** 2, axis=1, keepdims=True)
    gamma = jnp.reshape(bn_g_ref[...], (KSTEPS, 1))
    beta = jnp.reshape(bn_b_ref[...], (KSTEPS, 1))
    posT_n = (posT - mean) * jax.lax.rsqrt(var + 1e-5) * gamma + beta
    pe_featT = jax.lax.dot_general(pe_W_ref[...], posT_n,
                                   (((0,), (0,)), ((), ())),
                                   preferred_element_type=jnp.float32)
    pe_featT = pe_featT + pe_b_ref[...][:, None]

    deg2 = jnp.sum(A, axis=0) + 1.0
    dis = jax.lax.rsqrt(deg2)
    S = dis[:, None] * dis[None, :] * (A + eye)
    S_ref[...] = S.astype(jnp.bfloat16)

    W1b = W1_ref[D:2 * D, :]
    pe_contribT = jax.lax.dot_general(W1b, pe_featT,
                                      (((0,), (0,)), ((), ())),
                                      preferred_element_type=jnp.float32)
    c1T = jnp.dot(pe_contribT, S, preferred_element_type=jnp.float32)
    c1T = c1T + b1_ref[...][:, None]
    for g in range(12):
        c1T_ref[g * DFF:(g + 1) * DFF, :] = c1T


def _fused_body(xt_ref, cirmat_ref, bn_g_ref, bn_b_ref, pe_W_ref, pe_b_ref,
                W1_ref, b1_ref, W2t_ref, b2_ref, outT_ref,
                posT_ref, S_ref, c1T_ref):
    i = pl.program_id(0)

    @pl.when(i == 0)
    def _():
        _prep(cirmat_ref, bn_g_ref, bn_b_ref, pe_W_ref, pe_b_ref, W1_ref,
              b1_ref, S_ref, c1T_ref, posT_ref)

    @pl.when(i > 0)
    def _():
        W1a = W1_ref[0:D, :]
        W2t = W2t_ref[...]
        S = S_ref[...]
        t1s = [jax.lax.dot_general(W1a, xt_ref[g // 12, g % 12],
                                   (((0,), (0,)), ((), ())),
                                   preferred_element_type=jnp.float32)
               for g in range(NB * 12)]
        T1 = jnp.concatenate(t1s, axis=0)
        H1 = jnp.dot(T1.astype(jnp.bfloat16), S,
                     preferred_element_type=jnp.float32)
        c1 = c1T_ref[...]
        H1 = jnp.maximum(
            H1 + jnp.concatenate([c1] * NB, axis=0), 0.0)
        t2s = [jnp.dot(W2t, H1[g * DFF:(g + 1) * DFF, :],
                       preferred_element_type=jnp.float32)
               for g in range(NB * 12)]
        T2 = jnp.concatenate(t2s, axis=0)
        OUT = jnp.dot(T2.astype(jnp.bfloat16), S,
                      preferred_element_type=jnp.float32)
        outT_ref[...] = (jnp.reshape(OUT, (NB, 12, D, N))
                         + b2_ref[...][None, None, :, None])


def kernel(x, cirmat, bn_gamma, bn_beta, pe_W, pe_b, W1, b1, W2, b2):
    B, T, n, d = x.shape
    xt = jnp.transpose(x, (0, 1, 3, 2))
    W2t = jnp.transpose(W2)

    def xmap(i):
        j = jnp.maximum(i - 1, 0)
        return (j, 0, 0, 0)

    outT = pl.pallas_call(
        _fused_body,
        grid=(B // NB + 1,),
        in_specs=[
            pl.BlockSpec((NB, T, D, N), xmap),
            pl.BlockSpec((N, N), lambda i: (0, 0)),
            pl.BlockSpec((D,), lambda i: (0,)),
            pl.BlockSpec((D,), lambda i: (0,)),
            pl.BlockSpec((D, D), lambda i: (0, 0)),
            pl.BlockSpec((D,), lambda i: (0,)),
            pl.BlockSpec((2 * D, DFF), lambda i: (0, 0)),
            pl.BlockSpec((DFF,), lambda i: (0,)),
            pl.BlockSpec((D, DFF), lambda i: (0, 0)),
            pl.BlockSpec((D,), lambda i: (0,)),
        ],
        out_specs=pl.BlockSpec((NB, T, D, N), xmap),
        out_shape=jax.ShapeDtypeStruct((B, T, D, N), jnp.float32),
        scratch_shapes=[
            pltpu.VMEM((72, N), jnp.float32),
            pltpu.VMEM((N, N), jnp.bfloat16),
            pltpu.VMEM((12 * DFF, N), jnp.float32),
        ],
    )(xt, cirmat, bn_gamma, bn_beta, pe_W, pe_b, W1, b1, W2t, b2)
    return jnp.transpose(outT, (0, 1, 3, 2))


# speedup vs baseline: 1.0317x; 1.0317x over previous
import jax
import jax.numpy as jnp
from jax.experimental import pallas as pl
from jax.experimental.pallas import tpu as pltpu

N = 325
D = 64
DFF = 128
KSTEPS = 64
NB = 2


def _prep(cirmat_ref, bn_g_ref, bn_b_ref, pe_W_ref, pe_b_ref, W1_ref,
          b1_ref, S_ref, c1T_ref, posT_ref):
    A = cirmat_ref[...]
    rows = jax.lax.broadcasted_iota(jnp.int32, (N, N), 0)
    cols = jax.lax.broadcasted_iota(jnp.int32, (N, N), 1)
    eye = (rows == cols).astype(jnp.float32)

    deg = jnp.sum(A, axis=1)
    dinv = jnp.where(deg > 0, 1.0 / deg, 0.0)
    P = dinv[:, None] * A
    P2 = jnp.dot(P, P, preferred_element_type=jnp.float32)
    P3 = jnp.dot(P2, P, preferred_element_type=jnp.float32)
    P4 = jnp.dot(P2, P2, preferred_element_type=jnp.float32)
    P5 = jnp.dot(P4, P, preferred_element_type=jnp.float32)
    P6 = jnp.dot(P4, P2, preferred_element_type=jnp.float32)
    P7 = jnp.dot(P4, P3, preferred_element_type=jnp.float32)
    P8 = jnp.dot(P4, P4, preferred_element_type=jnp.float32)
    mats = [P, P2, P3, P4, P5, P6, P7]
    matsT = [jnp.transpose(M) for M in mats]
    posT_ref[0:8, :] = jnp.stack(
        [jnp.sum(M * eye, axis=0) for M in mats]
        + [jnp.sum(P8 * eye, axis=0)])
    R = P8
    for j in range(1, 8):
        R2 = jnp.dot(R, P8, preferred_element_type=jnp.float32)
        rows = ([jnp.sum(MT * R, axis=0) for MT in matsT]
                + [jnp.sum(eye * R2, axis=0)])
        posT_ref[8 * j:8 * j + 8, :] = jnp.stack(rows)
        R = R2
    posT = posT_ref[0:KSTEPS, :]

    mean = jnp.mean(posT, axis=1, keepdims=True)
    var = jnp.mean((posT - mean) ** 2, axis=1, keepdims=True)
    gamma = jnp.reshape(bn_g_ref[...], (KSTEPS, 1))
    beta = jnp.reshape(bn_b_ref[...], (KSTEPS, 1))
    posT_n = (posT - mean) * jax.lax.rsqrt(var + 1e-5) * gamma + beta
    pe_featT = jax.lax.dot_general(pe_W_ref[...], posT_n,
                                   (((0,), (0,)), ((), ())),
                                   preferred_element_type=jnp.float32)
    pe_featT = pe_featT + pe_b_ref[...][:, None]

    deg2 = jnp.sum(A, axis=0) + 1.0
    dis = jax.lax.rsqrt(deg2)
    S = dis[:, None] * dis[None, :] * (A + eye)
    S_ref[...] = S.astype(jnp.bfloat16)

    W1b = W1_ref[D:2 * D, :]
    pe_contribT = jax.lax.dot_general(W1b, pe_featT,
                                      (((0,), (0,)), ((), ())),
                                      preferred_element_type=jnp.float32)
    c1T = jnp.dot(pe_contribT, S, preferred_element_type=jnp.float32)
    c1T = c1T + b1_ref[...][:, None]
    for g in range(12):
        c1T_ref[g * DFF:(g + 1) * DFF, :] = c1T


def _fused_body(xt_ref, cirmat_ref, bn_g_ref, bn_b_ref, pe_W_ref, pe_b_ref,
                W1_ref, b1_ref, W2t_ref, b2_ref, outT_ref,
                posT_ref, S_ref, c1T_ref):
    i = pl.program_id(0)

    @pl.when(i == 0)
    def _():
        _prep(cirmat_ref, bn_g_ref, bn_b_ref, pe_W_ref, pe_b_ref, W1_ref,
              b1_ref, S_ref, c1T_ref, posT_ref)

    @pl.when(i > 0)
    def _():
        W1a = W1_ref[0:D, :]
        W2t = W2t_ref[...]
        S = S_ref[...]
        t1s = [jax.lax.dot_general(W1a, xt_ref[g // 12, g % 12],
                                   (((0,), (0,)), ((), ())),
                                   preferred_element_type=jnp.float32)
               for g in range(NB * 12)]
        T1 = jnp.concatenate(t1s, axis=0)
        H1 = jnp.dot(T1.astype(jnp.bfloat16), S,
                     preferred_element_type=jnp.float32)
        c1 = c1T_ref[...]
        H1 = jnp.maximum(
            H1 + jnp.concatenate([c1] * NB, axis=0), 0.0)
        t2s = [jnp.dot(W2t, H1[g * DFF:(g + 1) * DFF, :],
                       preferred_element_type=jnp.float32)
               for g in range(NB * 12)]
        T2 = jnp.concatenate(t2s, axis=0)
        OUT = jnp.dot(T2.astype(jnp.bfloat16), S,
                      preferred_element_type=jnp.float32)
        outT_ref[...] = (jnp.reshape(OUT, (NB, 12, D, N))
                         + b2_ref[...][None, None, :, None])


def kernel(x, cirmat, bn_gamma, bn_beta, pe_W, pe_b, W1, b1, W2, b2):
    B, T, n, d = x.shape
    xt = jnp.transpose(x, (0, 1, 3, 2))
    W2t = jnp.transpose(W2)

    def xmap(i):
        j = jnp.maximum(i - 1, 0)
        return (j, 0, 0, 0)

    outT = pl.pallas_call(
        _fused_body,
        grid=(B // NB + 1,),
        in_specs=[
            pl.BlockSpec((NB, T, D, N), xmap),
            pl.BlockSpec((N, N), lambda i: (0, 0)),
            pl.BlockSpec((D,), lambda i: (0,)),
            pl.BlockSpec((D,), lambda i: (0,)),
            pl.BlockSpec((D, D), lambda i: (0, 0)),
            pl.BlockSpec((D,), lambda i: (0,)),
            pl.BlockSpec((2 * D, DFF), lambda i: (0, 0)),
            pl.BlockSpec((DFF,), lambda i: (0,)),
            pl.BlockSpec((D, DFF), lambda i: (0, 0)),
            pl.BlockSpec((D,), lambda i: (0,)),
        ],
        out_specs=pl.BlockSpec((NB, T, D, N), xmap),
        out_shape=jax.ShapeDtypeStruct((B, T, D, N), jnp.float32),
        scratch_shapes=[
            pltpu.VMEM((72, N), jnp.float32),
            pltpu.VMEM((N, N), jnp.bfloat16),
            pltpu.VMEM((12 * DFF, N), jnp.float32),
        ],
    )(xt, cirmat, bn_gamma, bn_beta, pe_W, pe_b, W1, b1, W2t, b2)
    return jnp.transpose(outT, (0, 1, 3, 2))
